# Initial kernel scaffold; baseline (speedup 1.0000x reference)
import functools

import jax
import jax.numpy as jnp
from jax import lax
from jax.experimental import pallas as pl
from jax.experimental.pallas import tpu as pltpu
from jax.experimental.pallas import tpu_sc as plsc

NN = 10000
EE = 320000
FD = 128

NC = 2
NS = 16
NW = NC * NS
ET = EE // NW
KB = 80
NP = 10240
RPT = NP // NS

_HI = jax.lax.Precision.HIGHEST
_BN_INV = 1.0 / (1.0 + 1e-5) ** 0.5


def _pre_body(x_ref, w_ref, as_ref, ad_ref, h_ref, s_ref, d_ref):
    h = jnp.dot(x_ref[...], w_ref[...], precision=_HI,
                preferred_element_type=jnp.float32)
    h_ref[...] = h
    s_ref[...] = jnp.sum(h * as_ref[...], axis=1, keepdims=True)
    d_ref[...] = jnp.sum(h * ad_ref[...], axis=1, keepdims=True)


def _tc_pre(x, W, att_s, att_d):
    return pl.pallas_call(
        _pre_body,
        out_shape=[
            jax.ShapeDtypeStruct((NN, FD), jnp.float32),
            jax.ShapeDtypeStruct((NN, 1), jnp.float32),
            jax.ShapeDtypeStruct((NN, 1), jnp.float32),
        ],
    )(x, W, att_s, att_d)


def _den_col(denp):
    ones = jnp.ones((NW, 1), jnp.float32)
    return lax.dot_general(denp, ones, (((0,), (0,)), ((), ())),
                           precision=_HI, preferred_element_type=jnp.float32)


def _mid_body(p0_ref, p1_ref, denp_ref, s_ref, d_ref, h_ref, b_ref, g_ref,
              bt_ref, w2_ref, as_ref, ad_ref, h2_ref, s2_ref, d2_ref):
    e = s_ref[...] + d_ref[...]
    w_self = jnp.exp(jnp.where(e >= 0, e, 0.2 * e))
    den = _den_col(denp_ref[...]) + w_self + 1e-16
    agg = (p0_ref[...] + p1_ref[...] + w_self * h_ref[...]) / den
    y = (agg + b_ref[...]) * (g_ref[...] * _BN_INV) + bt_ref[...]
    z = jnp.maximum(y, 0.0)
    h2 = jnp.dot(z, w2_ref[...], precision=_HI,
                 preferred_element_type=jnp.float32)
    h2_ref[...] = h2
    s2_ref[...] = jnp.sum(h2 * as_ref[...], axis=1, keepdims=True)
    d2_ref[...] = jnp.sum(h2 * ad_ref[...], axis=1, keepdims=True)


def _tc_mid(p0, p1, denp, s1, d1, h1, b1, g, bt, W2, as2, ad2):
    return pl.pallas_call(
        _mid_body,
        out_shape=[
            jax.ShapeDtypeStruct((NN, FD), jnp.float32),
            jax.ShapeDtypeStruct((NN, 1), jnp.float32),
            jax.ShapeDtypeStruct((NN, 1), jnp.float32),
        ],
    )(p0, p1, denp, s1, d1, h1, b1, g, bt, W2, as2, ad2)


def _post_body(p0_ref, p1_ref, denp_ref, s_ref, d_ref, h_ref, b_ref, o_ref):
    e = s_ref[...] + d_ref[...]
    w_self = jnp.exp(jnp.where(e >= 0, e, 0.2 * e))
    den = _den_col(denp_ref[...]) + w_self + 1e-16
    o_ref[...] = (p0_ref[...] + p1_ref[...] + w_self * h_ref[...]) / den \
        + b_ref[...]


def _tc_post(p0, p1, denp, s2, d2, h2, b2):
    return pl.pallas_call(
        _post_body,
        out_shape=jax.ShapeDtypeStruct((NN, FD), jnp.float32),
    )(p0, p1, denp, s2, d2, h2, b2)


def _sc_body(src_hbm, dst_hbm, s_hbm, d_hbm, h_hbm, outp_hbm, denp_hbm,
             src_v, dst_v, s_v, d_v, w_v, den_v, rows_v, src_b, dst_b,
             out_sh, sem):
    cid = lax.axis_index("c")
    sid = lax.axis_index("s")
    wid = sid * NC + cid
    zeros16 = jnp.zeros((16,), jnp.float32)

    def _zr(k, carry):
        for j in range(FD // 16):
            rows_v[k, pl.ds(16 * j, 16)] = zeros16
        return carry
    lax.fori_loop(0, KB, _zr, 0)
    for t in range(RPT // KB):
        pltpu.sync_copy(rows_v, out_sh.at[pl.ds(sid * RPT + t * KB, KB)])

    def _zd(j, carry):
        den_v[pl.ds(16 * j, 16)] = zeros16
        return carry
    lax.fori_loop(0, NN // 16, _zd, 0)

    base = wid * ET
    pltpu.sync_copy(src_hbm.at[pl.ds(base, ET)], src_v)
    pltpu.sync_copy(dst_hbm.at[pl.ds(base, ET)], dst_v)
    pltpu.sync_copy(s_hbm, s_v)
    pltpu.sync_copy(d_hbm, d_v)

    def _sp(j, carry):
        sv = src_v[pl.ds(16 * j, 16)]
        dv = dst_v[pl.ds(16 * j, 16)]
        e = plsc.load_gather(s_v, [sv]) + plsc.load_gather(d_v, [dv])
        e = jnp.where(e >= 0, e, 0.2 * e)
        w = jnp.exp(e)
        w_v[pl.ds(16 * j, 16)] = w
        plsc.addupdate_scatter(den_v, [dv], w)
        return carry
    lax.fori_loop(0, ET // 16, _sp, 0)

    plsc.subcore_barrier()

    def _hp(b, carry):
        eb = b * KB
        for j in range(KB // 16):
            src_b[pl.ds(16 * j, 16)] = src_v[pl.ds(eb + 16 * j, 16)]
            dst_b[pl.ds(16 * j, 16)] = dst_v[pl.ds(eb + 16 * j, 16)]
        pltpu.async_copy(h_hbm.at[src_b], rows_v, sem).wait()

        def _scale(k, c2):
            wb = lax.broadcast(w_v[eb + k], (16,))
            for j in range(FD // 16):
                rows_v[k, pl.ds(16 * j, 16)] = \
                    rows_v[k, pl.ds(16 * j, 16)] * wb
            return c2
        lax.fori_loop(0, KB, _scale, 0)
        pltpu.sync_copy(rows_v, out_sh.at[dst_b], add=True)
        return carry
    lax.fori_loop(0, ET // KB, _hp, 0)

    pltpu.sync_copy(den_v, denp_hbm.at[pl.ds(wid * NN, NN)])
    plsc.subcore_barrier()
    pltpu.sync_copy(out_sh.at[pl.ds(sid * RPT, RPT)],
                    outp_hbm.at[pl.ds(cid * NP + sid * RPT, RPT)])


@functools.partial(
    pl.kernel,
    out_type=[
        jax.ShapeDtypeStruct((2 * NP, FD), jnp.float32),
        jax.ShapeDtypeStruct((NW * NN,), jnp.float32),
    ],
    mesh=plsc.VectorSubcoreMesh(core_axis_name="c", subcore_axis_name="s"),
    scratch_types=[
        pltpu.VMEM((ET,), jnp.int32),
        pltpu.VMEM((ET,), jnp.int32),
        pltpu.VMEM((NN,), jnp.float32),
        pltpu.VMEM((NN,), jnp.float32),
        pltpu.VMEM((ET,), jnp.float32),
        pltpu.VMEM((NN,), jnp.float32),
        pltpu.VMEM((KB, FD), jnp.float32),
        pltpu.VMEM((KB,), jnp.int32),
        pltpu.VMEM((KB,), jnp.int32),
        pltpu.VMEM_SHARED((NP, FD), jnp.float32),
        pltpu.SemaphoreType.DMA,
    ],
)
def _sc_edge(src_hbm, dst_hbm, s_hbm, d_hbm, h_hbm, outp_hbm, denp_hbm,
             *rest):
    _sc_body(src_hbm, dst_hbm, s_hbm, d_hbm, h_hbm, outp_hbm, denp_hbm,
             *rest)


def kernel(x, edge_index, W1, att_src1, att_dst1, b1, bn_gamma, bn_beta,
           W2, att_src2, att_dst2, b2):
    src = edge_index[0]
    dst = edge_index[1]
    b1r = b1.reshape(1, FD)
    gr = bn_gamma.reshape(1, FD)
    btr = bn_beta.reshape(1, FD)
    b2r = b2.reshape(1, FD)

    h1, s1, d1 = _tc_pre(x, W1, att_src1, att_dst1)
    outp1, denp1 = _sc_edge(src, dst, s1.reshape(NN), d1.reshape(NN), h1)
    h2, s2, d2 = _tc_mid(outp1[0:NN], outp1[NP:NP + NN],
                         denp1.reshape(NW, NN), s1, d1, h1,
                         b1r, gr, btr, W2, att_src2, att_dst2)
    outp2, denp2 = _sc_edge(src, dst, s2.reshape(NN), d2.reshape(NN), h2)
    return _tc_post(outp2[0:NN], outp2[NP:NP + NN],
                    denp2.reshape(NW, NN), s2, d2, h2, b2r)


# speedup vs baseline: 21.3893x; 21.3893x over previous
import functools

import jax
import jax.numpy as jnp
from jax import lax
from jax.experimental import pallas as pl
from jax.experimental.pallas import tpu as pltpu
from jax.experimental.pallas import tpu_sc as plsc

NN = 10000
EE = 320000
FD = 128

NC = 2
NS = 16
NW = NC * NS
ET = EE // NW
KB = 80
NP = 10240
RPT = NP // NS

_HI = jax.lax.Precision.HIGHEST
_BN_INV = 1.0 / (1.0 + 1e-5) ** 0.5


def _pre_body(x_ref, w_ref, as_ref, ad_ref, h_ref, s_ref, d_ref):
    h = jnp.dot(x_ref[...], w_ref[...], precision=_HI,
                preferred_element_type=jnp.float32)
    h_ref[...] = h
    s_ref[...] = jnp.sum(h * as_ref[...], axis=1, keepdims=True)
    d_ref[...] = jnp.sum(h * ad_ref[...], axis=1, keepdims=True)


_RB = 2000
_NG = NN // _RB


def _row_spec(width):
    return pl.BlockSpec((_RB, width), lambda i: (i, 0))


def _full_spec(r, c):
    return pl.BlockSpec((r, c), lambda i: (0, 0))


def _tc_pre(x, W, att_s, att_d):
    return pl.pallas_call(
        _pre_body,
        grid=(_NG,),
        in_specs=[_row_spec(FD), _full_spec(FD, FD), _full_spec(1, FD),
                  _full_spec(1, FD)],
        out_specs=[_row_spec(FD), _row_spec(1), _row_spec(1)],
        out_shape=[
            jax.ShapeDtypeStruct((NN, FD), jnp.float32),
            jax.ShapeDtypeStruct((NN, 1), jnp.float32),
            jax.ShapeDtypeStruct((NN, 1), jnp.float32),
        ],
    )(x, W, att_s, att_d)


def _den_col(denp_ref):
    dp = denp_ref[0]
    ones = jnp.ones((NW, 1), jnp.float32)
    return lax.dot_general(dp, ones, (((0,), (0,)), ((), ())),
                           precision=_HI, preferred_element_type=jnp.float32)


def _mid_body(p0_ref, p1_ref, denp_ref, s_ref, d_ref, h_ref, b_ref, g_ref,
              bt_ref, w2_ref, as_ref, ad_ref, h2_ref, s2_ref, d2_ref):
    e = s_ref[...] + d_ref[...]
    w_self = jnp.exp(jnp.where(e >= 0, e, 0.2 * e))
    den = _den_col(denp_ref) + w_self + 1e-16
    agg = (p0_ref[...] + p1_ref[...] + w_self * h_ref[...]) / den
    y = (agg + b_ref[...]) * (g_ref[...] * _BN_INV) + bt_ref[...]
    z = jnp.maximum(y, 0.0)
    h2 = jnp.dot(z, w2_ref[...], precision=_HI,
                 preferred_element_type=jnp.float32)
    h2_ref[...] = h2
    s2_ref[...] = jnp.sum(h2 * as_ref[...], axis=1, keepdims=True)
    d2_ref[...] = jnp.sum(h2 * ad_ref[...], axis=1, keepdims=True)


def _tc_mid(p0, p1, denp, s1, d1, h1, b1, g, bt, W2, as2, ad2):
    return pl.pallas_call(
        _mid_body,
        grid=(_NG,),
        in_specs=[_row_spec(FD), _row_spec(FD),
                  pl.BlockSpec((1, NW, _RB), lambda i: (i, 0, 0)),
                  _row_spec(1), _row_spec(1), _row_spec(FD),
                  _full_spec(1, FD), _full_spec(1, FD), _full_spec(1, FD),
                  _full_spec(FD, FD), _full_spec(1, FD), _full_spec(1, FD)],
        out_specs=[_row_spec(FD), _row_spec(1), _row_spec(1)],
        out_shape=[
            jax.ShapeDtypeStruct((NN, FD), jnp.float32),
            jax.ShapeDtypeStruct((NN, 1), jnp.float32),
            jax.ShapeDtypeStruct((NN, 1), jnp.float32),
        ],
    )(p0, p1, denp, s1, d1, h1, b1, g, bt, W2, as2, ad2)


def _post_body(p0_ref, p1_ref, denp_ref, s_ref, d_ref, h_ref, b_ref, o_ref):
    e = s_ref[...] + d_ref[...]
    w_self = jnp.exp(jnp.where(e >= 0, e, 0.2 * e))
    den = _den_col(denp_ref) + w_self + 1e-16
    o_ref[...] = (p0_ref[...] + p1_ref[...] + w_self * h_ref[...]) / den \
        + b_ref[...]


def _tc_post(p0, p1, denp, s2, d2, h2, b2):
    return pl.pallas_call(
        _post_body,
        grid=(_NG,),
        in_specs=[_row_spec(FD), _row_spec(FD),
                  pl.BlockSpec((1, NW, _RB), lambda i: (i, 0, 0)),
                  _row_spec(1), _row_spec(1), _row_spec(FD),
                  _full_spec(1, FD)],
        out_specs=_row_spec(FD),
        out_shape=jax.ShapeDtypeStruct((NN, FD), jnp.float32),
    )(p0, p1, denp, s2, d2, h2, b2)


def _sc_body(src_hbm, dst_hbm, s_hbm, d_hbm, h_hbm, outp_hbm, denp_hbm,
             s_v, d_v, den_v, rows_v, src_b, dst_b, w_b, out_sh, sem):
    cid = lax.axis_index("c")
    sid = lax.axis_index("s")
    wid = sid * NC + cid
    zeros16 = jnp.zeros((16,), jnp.float32)

    def _zr(k, carry):
        for j in range(FD // 16):
            rows_v[k, pl.ds(16 * j, 16)] = zeros16
        return carry
    lax.fori_loop(0, KB, _zr, 0)
    for t in range(RPT // KB):
        pltpu.sync_copy(rows_v, out_sh.at[pl.ds(sid * RPT + t * KB, KB)])

    def _zd(j, carry):
        den_v[pl.ds(16 * j, 16)] = zeros16
        return carry
    lax.fori_loop(0, NN // 16, _zd, 0)

    pltpu.sync_copy(s_hbm, s_v)
    pltpu.sync_copy(d_hbm, d_v)
    plsc.subcore_barrier()

    base = wid * ET

    def _hp(b, carry):
        eb = base + b * KB
        pltpu.sync_copy(src_hbm.at[pl.ds(eb, KB)], src_b)
        pltpu.sync_copy(dst_hbm.at[pl.ds(eb, KB)], dst_b)
        for j in range(KB // 16):
            sv = src_b[pl.ds(16 * j, 16)]
            dv = dst_b[pl.ds(16 * j, 16)]
            e = plsc.load_gather(s_v, [sv]) + plsc.load_gather(d_v, [dv])
            e = jnp.where(e >= 0, e, 0.2 * e)
            w = jnp.exp(e)
            w_b[pl.ds(16 * j, 16)] = w
            plsc.addupdate_scatter(den_v, [dv], w)
        pltpu.async_copy(h_hbm.at[src_b], rows_v, sem).wait()

        def _scale(k, c2):
            wb = plsc.load_gather(w_b, [lax.broadcast(k, (16,))])
            for j in range(FD // 16):
                rows_v[k, pl.ds(16 * j, 16)] = \
                    rows_v[k, pl.ds(16 * j, 16)] * wb
            return c2
        lax.fori_loop(0, KB, _scale, 0)
        pltpu.sync_copy(rows_v, out_sh.at[dst_b], add=True)
        return carry
    lax.fori_loop(0, ET // KB, _hp, 0)

    pltpu.sync_copy(den_v, denp_hbm.at[pl.ds(wid * NN, NN)])
    plsc.subcore_barrier()
    pltpu.sync_copy(out_sh.at[pl.ds(sid * RPT, RPT)],
                    outp_hbm.at[pl.ds(cid * NP + sid * RPT, RPT)])


@functools.lru_cache(maxsize=None)
def _get_sc_edge():
    return pl.kernel(
        _sc_body,
        out_type=[
            jax.ShapeDtypeStruct((2 * NP, FD), jnp.float32),
            jax.ShapeDtypeStruct((NW * NN,), jnp.float32),
        ],
        mesh=plsc.VectorSubcoreMesh(core_axis_name="c", subcore_axis_name="s",
                                    num_cores=NC, num_subcores=NS),
        scratch_types=[
            pltpu.VMEM((NN,), jnp.float32),
            pltpu.VMEM((NN,), jnp.float32),
            pltpu.VMEM((NN,), jnp.float32),
            pltpu.VMEM((KB, FD), jnp.float32),
            pltpu.VMEM((KB,), jnp.int32),
            pltpu.VMEM((KB,), jnp.int32),
            pltpu.VMEM((KB,), jnp.float32),
            pltpu.VMEM_SHARED((NP, FD), jnp.float32),
            pltpu.SemaphoreType.DMA,
        ],
        compiler_params=pltpu.CompilerParams(needs_layout_passes=False),
    )


def kernel(x, edge_index, W1, att_src1, att_dst1, b1, bn_gamma, bn_beta,
           W2, att_src2, att_dst2, b2):
    src = edge_index[0]
    dst = edge_index[1]
    b1r = b1.reshape(1, FD)
    gr = bn_gamma.reshape(1, FD)
    btr = bn_beta.reshape(1, FD)
    b2r = b2.reshape(1, FD)

    h1, s1, d1 = _tc_pre(x, W1, att_src1, att_dst1)
    outp1, denp1 = _get_sc_edge()(src, dst, s1.reshape(NN), d1.reshape(NN), h1)
    dp1 = denp1.reshape(NW, _NG, _RB).transpose(1, 0, 2)
    h2, s2, d2 = _tc_mid(outp1[0:NN], outp1[NP:NP + NN], dp1, s1, d1, h1,
                         b1r, gr, btr, W2, att_src2, att_dst2)
    outp2, denp2 = _get_sc_edge()(src, dst, s2.reshape(NN), d2.reshape(NN), h2)
    dp2 = denp2.reshape(NW, _NG, _RB).transpose(1, 0, 2)
    return _tc_post(outp2[0:NN], outp2[NP:NP + NN], dp2, s2, d2, h2, b2r)


# speedup vs baseline: 34.9393x; 1.6335x over previous
import functools

import jax
import jax.numpy as jnp
from jax import lax
from jax.experimental import pallas as pl
from jax.experimental.pallas import tpu as pltpu
from jax.experimental.pallas import tpu_sc as plsc

NN = 10000
EE = 320000
FD = 128

NC = 2
NS = 16
NW = NC * NS
ET = EE // NW
KB = 80
SG = 5
CW = KB * SG
NSG = EE // (NC * NS) // CW
NP = 10240
RPT = NP // NS

_HI = jax.lax.Precision.HIGHEST
_BN_INV = 1.0 / (1.0 + 1e-5) ** 0.5


def _pre_body(x_ref, w_ref, as_ref, ad_ref, h_ref, s_ref, d_ref):
    h = jnp.dot(x_ref[...], w_ref[...], precision=_HI,
                preferred_element_type=jnp.float32)
    h_ref[...] = h
    s_ref[...] = jnp.sum(h * as_ref[...], axis=1, keepdims=True)
    d_ref[...] = jnp.sum(h * ad_ref[...], axis=1, keepdims=True)


_RB = 2000
_NG = NN // _RB


def _row_spec(width):
    return pl.BlockSpec((_RB, width), lambda i: (i, 0))


def _full_spec(r, c):
    return pl.BlockSpec((r, c), lambda i: (0, 0))


def _tc_pre(x, W, att_s, att_d):
    return pl.pallas_call(
        _pre_body,
        grid=(_NG,),
        in_specs=[_row_spec(FD), _full_spec(FD, FD), _full_spec(1, FD),
                  _full_spec(1, FD)],
        out_specs=[_row_spec(FD), _row_spec(1), _row_spec(1)],
        out_shape=[
            jax.ShapeDtypeStruct((NN, FD), jnp.float32),
            jax.ShapeDtypeStruct((NN, 1), jnp.float32),
            jax.ShapeDtypeStruct((NN, 1), jnp.float32),
        ],
    )(x, W, att_s, att_d)


def _den_col(denp_ref):
    dp = denp_ref[0]
    ones = jnp.ones((NC, 1), jnp.float32)
    return lax.dot_general(dp, ones, (((0,), (0,)), ((), ())),
                           precision=_HI, preferred_element_type=jnp.float32)


def _mid_body(p0_ref, p1_ref, denp_ref, s_ref, d_ref, h_ref, b_ref, g_ref,
              bt_ref, w2_ref, as_ref, ad_ref, h2_ref, s2_ref, d2_ref):
    e = s_ref[...] + d_ref[...]
    w_self = jnp.exp(jnp.where(e >= 0, e, 0.2 * e))
    den = _den_col(denp_ref) + w_self + 1e-16
    agg = (p0_ref[...] + p1_ref[...] + w_self * h_ref[...]) / den
    y = (agg + b_ref[...]) * (g_ref[...] * _BN_INV) + bt_ref[...]
    z = jnp.maximum(y, 0.0)
    h2 = jnp.dot(z, w2_ref[...], precision=_HI,
                 preferred_element_type=jnp.float32)
    h2_ref[...] = h2
    s2_ref[...] = jnp.sum(h2 * as_ref[...], axis=1, keepdims=True)
    d2_ref[...] = jnp.sum(h2 * ad_ref[...], axis=1, keepdims=True)


def _tc_mid(p0, p1, denp, s1, d1, h1, b1, g, bt, W2, as2, ad2):
    return pl.pallas_call(
        _mid_body,
        grid=(_NG,),
        in_specs=[_row_spec(FD), _row_spec(FD),
                  pl.BlockSpec((1, NC, _RB), lambda i: (i, 0, 0)),
                  _row_spec(1), _row_spec(1), _row_spec(FD),
                  _full_spec(1, FD), _full_spec(1, FD), _full_spec(1, FD),
                  _full_spec(FD, FD), _full_spec(1, FD), _full_spec(1, FD)],
        out_specs=[_row_spec(FD), _row_spec(1), _row_spec(1)],
        out_shape=[
            jax.ShapeDtypeStruct((NN, FD), jnp.float32),
            jax.ShapeDtypeStruct((NN, 1), jnp.float32),
            jax.ShapeDtypeStruct((NN, 1), jnp.float32),
        ],
    )(p0, p1, denp, s1, d1, h1, b1, g, bt, W2, as2, ad2)


def _post_body(p0_ref, p1_ref, denp_ref, s_ref, d_ref, h_ref, b_ref, o_ref):
    e = s_ref[...] + d_ref[...]
    w_self = jnp.exp(jnp.where(e >= 0, e, 0.2 * e))
    den = _den_col(denp_ref) + w_self + 1e-16
    o_ref[...] = (p0_ref[...] + p1_ref[...] + w_self * h_ref[...]) / den \
        + b_ref[...]


def _tc_post(p0, p1, denp, s2, d2, h2, b2):
    return pl.pallas_call(
        _post_body,
        grid=(_NG,),
        in_specs=[_row_spec(FD), _row_spec(FD),
                  pl.BlockSpec((1, NC, _RB), lambda i: (i, 0, 0)),
                  _row_spec(1), _row_spec(1), _row_spec(FD),
                  _full_spec(1, FD)],
        out_specs=_row_spec(FD),
        out_shape=jax.ShapeDtypeStruct((NN, FD), jnp.float32),
    )(p0, p1, denp, s2, d2, h2, b2)


def _sc_body(src_hbm, dst_hbm, s_hbm, d_hbm, h_hbm, outp_hbm, denp_hbm,
             s_v, d_v, rows0, rows1, src_g, dst_g, dst_b0, dst_b1,
             w_b0, w_b1, zbuf, out_sh, den_sh, sem_r0, sem_r1, sem_den):
    cid = lax.axis_index("c")
    sid = lax.axis_index("s")
    wid = sid * NC + cid
    zeros16 = jnp.zeros((16,), jnp.float32)
    izeros16 = jnp.zeros((16,), jnp.int32)
    rows = (rows0, rows1)
    dst_b = (dst_b0, dst_b1)
    w_b = (w_b0, w_b1)
    sem_r = (sem_r0, sem_r1)
    DSL = NP // NS

    def _zr(k, carry):
        for j in range(FD // 16):
            rows0[k, pl.ds(16 * j, 16)] = zeros16
        return carry
    lax.fori_loop(0, KB, _zr, 0)
    for t in range(RPT // KB):
        pltpu.sync_copy(rows0, out_sh.at[pl.ds(sid * RPT + t * KB, KB)])

    def _zz(j, carry):
        zbuf[pl.ds(16 * j, 16)] = zeros16
        return carry
    lax.fori_loop(0, DSL // 16, _zz, 0)
    pltpu.sync_copy(zbuf, den_sh.at[pl.ds(sid * DSL, DSL)])

    for j in range(KB // 16):
        dst_b0[pl.ds(16 * j, 16)] = izeros16
        dst_b1[pl.ds(16 * j, 16)] = izeros16
        w_b0[pl.ds(16 * j, 16)] = zeros16
        w_b1[pl.ds(16 * j, 16)] = zeros16

    pltpu.sync_copy(s_hbm, s_v)
    pltpu.sync_copy(d_hbm, d_v)
    plsc.subcore_barrier()

    pltpu.async_copy(w_b0, den_sh.at[dst_b0], sem_den, add=True)
    pltpu.async_copy(w_b1, den_sh.at[dst_b1], sem_den, add=True)

    base = wid * ET

    def _sg(G, carry):
        cb = base + G * CW
        pltpu.sync_copy(src_hbm.at[pl.ds(cb, CW)], src_g)
        pltpu.sync_copy(dst_hbm.at[pl.ds(cb, CW)], dst_g)
        pltpu.async_copy(h_hbm.at[src_g.at[pl.ds(0, KB)]], rows0, sem_r0)
        for k in range(SG):
            q = k % 2
            pltpu.make_async_copy(w_b[q], den_sh.at[dst_b[q]],
                                  sem_den).wait()
            for j in range(KB // 16):
                sv = src_g[pl.ds(k * KB + 16 * j, 16)]
                dv = dst_g[pl.ds(k * KB + 16 * j, 16)]
                e = plsc.load_gather(s_v, [sv]) + plsc.load_gather(d_v, [dv])
                e = jnp.where(e >= 0, e, 0.2 * e)
                w_b[q][pl.ds(16 * j, 16)] = jnp.exp(e)
                dst_b[q][pl.ds(16 * j, 16)] = dv
            pltpu.async_copy(w_b[q], den_sh.at[dst_b[q]], sem_den, add=True)
            if k < SG - 1:
                pltpu.async_copy(
                    h_hbm.at[src_g.at[pl.ds((k + 1) * KB, KB)]],
                    rows[1 - q], sem_r[1 - q])
            pltpu.make_async_copy(h_hbm.at[src_g.at[pl.ds(k * KB, KB)]],
                                  rows[q], sem_r[q]).wait()

            def _scale(kk, c2, _q=q):
                wb = plsc.load_gather(w_b[_q], [lax.broadcast(kk, (16,))])
                for j in range(FD // 16):
                    rows[_q][kk, pl.ds(16 * j, 16)] = \
                        rows[_q][kk, pl.ds(16 * j, 16)] * wb
                return c2
            lax.fori_loop(0, KB, _scale, 0, unroll=8)
            pltpu.sync_copy(rows[q], out_sh.at[dst_b[q]], add=True)
        return carry
    lax.fori_loop(0, NSG, _sg, 0)

    pltpu.make_async_copy(w_b0, den_sh.at[dst_b0], sem_den).wait()
    pltpu.make_async_copy(w_b1, den_sh.at[dst_b1], sem_den).wait()
    plsc.subcore_barrier()
    pltpu.sync_copy(den_sh.at[pl.ds(sid * DSL, DSL)],
                    denp_hbm.at[pl.ds(cid * NP + sid * DSL, DSL)])
    pltpu.sync_copy(out_sh.at[pl.ds(sid * RPT, RPT)],
                    outp_hbm.at[pl.ds(cid * NP + sid * RPT, RPT)])


@functools.lru_cache(maxsize=None)
def _get_sc_edge():
    return pl.kernel(
        _sc_body,
        out_type=[
            jax.ShapeDtypeStruct((2 * NP, FD), jnp.float32),
            jax.ShapeDtypeStruct((2 * NP,), jnp.float32),
        ],
        mesh=plsc.VectorSubcoreMesh(core_axis_name="c", subcore_axis_name="s",
                                    num_cores=NC, num_subcores=NS),
        scratch_types=[
            pltpu.VMEM((NN,), jnp.float32),
            pltpu.VMEM((NN,), jnp.float32),
            pltpu.VMEM((KB, FD), jnp.float32),
            pltpu.VMEM((KB, FD), jnp.float32),
            pltpu.VMEM((CW,), jnp.int32),
            pltpu.VMEM((CW,), jnp.int32),
            pltpu.VMEM((KB,), jnp.int32),
            pltpu.VMEM((KB,), jnp.int32),
            pltpu.VMEM((KB,), jnp.float32),
            pltpu.VMEM((KB,), jnp.float32),
            pltpu.VMEM((NP // NS,), jnp.float32),
            pltpu.VMEM_SHARED((NP, FD), jnp.float32),
            pltpu.VMEM_SHARED((NP,), jnp.float32),
            pltpu.SemaphoreType.DMA,
            pltpu.SemaphoreType.DMA,
            pltpu.SemaphoreType.DMA,
        ],
        compiler_params=pltpu.CompilerParams(needs_layout_passes=False),
    )


def _den_stack(denp):
    dp = jnp.stack([denp[0:NN], denp[NP:NP + NN]])
    return dp.reshape(NC, _NG, _RB).transpose(1, 0, 2)


def kernel(x, edge_index, W1, att_src1, att_dst1, b1, bn_gamma, bn_beta,
           W2, att_src2, att_dst2, b2):
    src = edge_index[0]
    dst = edge_index[1]
    b1r = b1.reshape(1, FD)
    gr = bn_gamma.reshape(1, FD)
    btr = bn_beta.reshape(1, FD)
    b2r = b2.reshape(1, FD)

    h1, s1, d1 = _tc_pre(x, W1, att_src1, att_dst1)
    outp1, denp1 = _get_sc_edge()(src, dst, s1.reshape(NN), d1.reshape(NN), h1)
    dp1 = _den_stack(denp1)
    h2, s2, d2 = _tc_mid(outp1[0:NN], outp1[NP:NP + NN], dp1, s1, d1, h1,
                         b1r, gr, btr, W2, att_src2, att_dst2)
    outp2, denp2 = _get_sc_edge()(src, dst, s2.reshape(NN), d2.reshape(NN), h2)
    dp2 = _den_stack(denp2)
    return _tc_post(outp2[0:NN], outp2[NP:NP + NN], dp2, s2, d2, h2, b2r)


# speedup vs baseline: 35.8304x; 1.0255x over previous
import functools

import jax
import jax.numpy as jnp
from jax import lax
from jax.experimental import pallas as pl
from jax.experimental.pallas import tpu as pltpu
from jax.experimental.pallas import tpu_sc as plsc

NN = 10000
EE = 320000
FD = 128

NC = 2
NS = 16
NW = NC * NS
ET = EE // NW
KB = 80
SG = 4
CW = KB * SG
NSG = 31
NP = 10240
RPT = NP // NS

_HI = jax.lax.Precision.HIGHEST
_BN_INV = 1.0 / (1.0 + 1e-5) ** 0.5


def _pre_body(x_ref, w_ref, as_ref, ad_ref, h_ref, s_ref, d_ref):
    h = jnp.dot(x_ref[...], w_ref[...], precision=_HI,
                preferred_element_type=jnp.float32)
    h_ref[...] = h
    s_ref[...] = jnp.sum(h * as_ref[...], axis=1, keepdims=True)
    d_ref[...] = jnp.sum(h * ad_ref[...], axis=1, keepdims=True)


_RB = 2000
_NG = NN // _RB


def _row_spec(width):
    return pl.BlockSpec((_RB, width), lambda i: (i, 0))


def _full_spec(r, c):
    return pl.BlockSpec((r, c), lambda i: (0, 0))


def _tc_pre(x, W, att_s, att_d):
    return pl.pallas_call(
        _pre_body,
        grid=(_NG,),
        in_specs=[_row_spec(FD), _full_spec(FD, FD), _full_spec(1, FD),
                  _full_spec(1, FD)],
        out_specs=[_row_spec(FD), _row_spec(1), _row_spec(1)],
        out_shape=[
            jax.ShapeDtypeStruct((NN, FD), jnp.float32),
            jax.ShapeDtypeStruct((NN, 1), jnp.float32),
            jax.ShapeDtypeStruct((NN, 1), jnp.float32),
        ],
    )(x, W, att_s, att_d)


def _den_col(denp_ref):
    dp = denp_ref[0]
    ones = jnp.ones((NC, 1), jnp.float32)
    return lax.dot_general(dp, ones, (((0,), (0,)), ((), ())),
                           precision=_HI, preferred_element_type=jnp.float32)


def _mid_body(p0_ref, p1_ref, denp_ref, s_ref, d_ref, h_ref, b_ref, g_ref,
              bt_ref, w2_ref, as_ref, ad_ref, h2_ref, s2_ref, d2_ref):
    e = s_ref[...] + d_ref[...]
    w_self = jnp.exp(jnp.where(e >= 0, e, 0.2 * e))
    den = _den_col(denp_ref) + w_self + 1e-16
    agg = (p0_ref[...] + p1_ref[...] + w_self * h_ref[...]) / den
    y = (agg + b_ref[...]) * (g_ref[...] * _BN_INV) + bt_ref[...]
    z = jnp.maximum(y, 0.0)
    h2 = jnp.dot(z, w2_ref[...], precision=_HI,
                 preferred_element_type=jnp.float32)
    h2_ref[...] = h2
    s2_ref[...] = jnp.sum(h2 * as_ref[...], axis=1, keepdims=True)
    d2_ref[...] = jnp.sum(h2 * ad_ref[...], axis=1, keepdims=True)


def _tc_mid(p0, p1, denp, s1, d1, h1, b1, g, bt, W2, as2, ad2):
    return pl.pallas_call(
        _mid_body,
        grid=(_NG,),
        in_specs=[_row_spec(FD), _row_spec(FD),
                  pl.BlockSpec((1, NC, _RB), lambda i: (i, 0, 0)),
                  _row_spec(1), _row_spec(1), _row_spec(FD),
                  _full_spec(1, FD), _full_spec(1, FD), _full_spec(1, FD),
                  _full_spec(FD, FD), _full_spec(1, FD), _full_spec(1, FD)],
        out_specs=[_row_spec(FD), _row_spec(1), _row_spec(1)],
        out_shape=[
            jax.ShapeDtypeStruct((NN, FD), jnp.float32),
            jax.ShapeDtypeStruct((NN, 1), jnp.float32),
            jax.ShapeDtypeStruct((NN, 1), jnp.float32),
        ],
    )(p0, p1, denp, s1, d1, h1, b1, g, bt, W2, as2, ad2)


def _post_body(p0_ref, p1_ref, denp_ref, s_ref, d_ref, h_ref, b_ref, o_ref):
    e = s_ref[...] + d_ref[...]
    w_self = jnp.exp(jnp.where(e >= 0, e, 0.2 * e))
    den = _den_col(denp_ref) + w_self + 1e-16
    o_ref[...] = (p0_ref[...] + p1_ref[...] + w_self * h_ref[...]) / den \
        + b_ref[...]


def _tc_post(p0, p1, denp, s2, d2, h2, b2):
    return pl.pallas_call(
        _post_body,
        grid=(_NG,),
        in_specs=[_row_spec(FD), _row_spec(FD),
                  pl.BlockSpec((1, NC, _RB), lambda i: (i, 0, 0)),
                  _row_spec(1), _row_spec(1), _row_spec(FD),
                  _full_spec(1, FD)],
        out_specs=_row_spec(FD),
        out_shape=jax.ShapeDtypeStruct((NN, FD), jnp.float32),
    )(p0, p1, denp, s2, d2, h2, b2)


def _sc_body(src_hbm, dst_hbm, s_hbm, d_hbm, h_hbm, outp_hbm, denp_hbm,
             s_v, d_v, rows0, rows1, src_g, dst_g, dst_b0, dst_b1,
             w_b0, w_b1, zbuf, out_sh, den_sh, sem_r0, sem_r1, sem_den,
             sem_sc):
    cid = lax.axis_index("c")
    sid = lax.axis_index("s")
    wid = sid * NC + cid
    zeros16 = jnp.zeros((16,), jnp.float32)
    izeros16 = jnp.zeros((16,), jnp.int32)
    rows = (rows0, rows1)
    dst_b = (dst_b0, dst_b1)
    w_b = (w_b0, w_b1)
    sem_r = (sem_r0, sem_r1)
    DSL = NP // NS

    def _zr(k, carry):
        for j in range(FD // 16):
            rows0[k, pl.ds(16 * j, 16)] = zeros16
            rows1[k, pl.ds(16 * j, 16)] = zeros16
        return carry
    lax.fori_loop(0, KB, _zr, 0)
    for t in range(RPT // KB):
        pltpu.sync_copy(rows0, out_sh.at[pl.ds(sid * RPT + t * KB, KB)])

    def _zz(j, carry):
        zbuf[pl.ds(16 * j, 16)] = zeros16
        return carry
    lax.fori_loop(0, DSL // 16, _zz, 0)
    pltpu.sync_copy(zbuf, den_sh.at[pl.ds(sid * DSL, DSL)])

    for j in range(KB // 16):
        dst_b0[pl.ds(16 * j, 16)] = izeros16
        dst_b1[pl.ds(16 * j, 16)] = izeros16
        w_b0[pl.ds(16 * j, 16)] = zeros16
        w_b1[pl.ds(16 * j, 16)] = zeros16

    pltpu.sync_copy(s_hbm, s_v)
    pltpu.sync_copy(d_hbm, d_v)
    plsc.subcore_barrier()

    pltpu.async_copy(w_b0, den_sh.at[dst_b0], sem_den, add=True)
    pltpu.async_copy(w_b1, den_sh.at[dst_b1], sem_den, add=True)

    base = wid * ET

    pltpu.async_copy(rows1, out_sh.at[dst_b1], sem_sc, add=True)
    pltpu.sync_copy(src_hbm.at[pl.ds(base, CW)], src_g)
    pltpu.sync_copy(dst_hbm.at[pl.ds(base, CW)], dst_g)
    pltpu.async_copy(h_hbm.at[src_g.at[pl.ds(0, KB)]], rows0, sem_r0)

    def _phase(k, q):
        pltpu.make_async_copy(w_b[q], den_sh.at[dst_b[q]], sem_den).wait()
        for j in range(KB // 16):
            sv = src_g[pl.ds(k * KB + 16 * j, 16)]
            dv = dst_g[pl.ds(k * KB + 16 * j, 16)]
            e = plsc.load_gather(s_v, [sv]) + plsc.load_gather(d_v, [dv])
            e = jnp.where(e >= 0, e, 0.2 * e)
            w_b[q][pl.ds(16 * j, 16)] = jnp.exp(e)
            dst_b[q][pl.ds(16 * j, 16)] = dv
        pltpu.async_copy(w_b[q], den_sh.at[dst_b[q]], sem_den, add=True)
        pltpu.make_async_copy(rows[1 - q], out_sh.at[dst_b[1 - q]],
                              sem_sc).wait()
        if k < SG - 1:
            pltpu.async_copy(h_hbm.at[src_g.at[pl.ds((k + 1) * KB, KB)]],
                             rows[1 - q], sem_r[1 - q])
        pltpu.make_async_copy(h_hbm.at[src_g.at[pl.ds(k * KB, KB)]],
                              rows[q], sem_r[q]).wait()

        def _scale(kk, c2, _q=q):
            wb = plsc.load_gather(w_b[_q], [lax.broadcast(kk, (16,))])
            for j in range(FD // 16):
                rows[_q][kk, pl.ds(16 * j, 16)] = \
                    rows[_q][kk, pl.ds(16 * j, 16)] * wb
            return c2
        lax.fori_loop(0, KB, _scale, 0, unroll=8)
        pltpu.async_copy(rows[q], out_sh.at[dst_b[q]], sem_sc, add=True)

    def _sg(G, carry):
        for k in range(SG):
            _phase(k, k % 2)

        @pl.when(G < NSG - 1)
        def _next_chunk():
            cb = base + (G + 1) * CW
            pltpu.sync_copy(src_hbm.at[pl.ds(cb, CW)], src_g)
            pltpu.sync_copy(dst_hbm.at[pl.ds(cb, CW)], dst_g)
            pltpu.async_copy(h_hbm.at[src_g.at[pl.ds(0, KB)]], rows0,
                             sem_r0)
        return carry
    lax.fori_loop(0, NSG, _sg, 0)

    tb = base + NSG * CW
    pltpu.make_async_copy(w_b0, den_sh.at[dst_b0], sem_den).wait()
    pltpu.sync_copy(src_hbm.at[pl.ds(tb, KB)], src_g.at[pl.ds(0, KB)])
    pltpu.sync_copy(dst_hbm.at[pl.ds(tb, KB)], dst_g.at[pl.ds(0, KB)])
    for j in range(KB // 16):
        sv = src_g[pl.ds(16 * j, 16)]
        dv = dst_g[pl.ds(16 * j, 16)]
        e = plsc.load_gather(s_v, [sv]) + plsc.load_gather(d_v, [dv])
        e = jnp.where(e >= 0, e, 0.2 * e)
        w_b0[pl.ds(16 * j, 16)] = jnp.exp(e)
        dst_b0[pl.ds(16 * j, 16)] = dv
    pltpu.async_copy(w_b0, den_sh.at[dst_b0], sem_den, add=True)
    pltpu.make_async_copy(rows1, out_sh.at[dst_b1], sem_sc).wait()
    pltpu.async_copy(h_hbm.at[src_g.at[pl.ds(0, KB)]], rows0, sem_r0)
    pltpu.make_async_copy(h_hbm.at[src_g.at[pl.ds(0, KB)]], rows0,
                          sem_r0).wait()

    def _scale_t(kk, c2):
        wb = plsc.load_gather(w_b0, [lax.broadcast(kk, (16,))])
        for j in range(FD // 16):
            rows0[kk, pl.ds(16 * j, 16)] = rows0[kk, pl.ds(16 * j, 16)] * wb
        return c2
    lax.fori_loop(0, KB, _scale_t, 0, unroll=8)
    pltpu.sync_copy(rows0, out_sh.at[dst_b0], add=True)

    pltpu.make_async_copy(w_b1, den_sh.at[dst_b1], sem_den).wait()
    pltpu.make_async_copy(w_b0, den_sh.at[dst_b0], sem_den).wait()
    plsc.subcore_barrier()
    pltpu.sync_copy(den_sh.at[pl.ds(sid * DSL, DSL)],
                    denp_hbm.at[pl.ds(cid * NP + sid * DSL, DSL)])
    pltpu.sync_copy(out_sh.at[pl.ds(sid * RPT, RPT)],
                    outp_hbm.at[pl.ds(cid * NP + sid * RPT, RPT)])


@functools.lru_cache(maxsize=None)
def _get_sc_edge():
    return pl.kernel(
        _sc_body,
        out_type=[
            jax.ShapeDtypeStruct((2 * NP, FD), jnp.float32),
            jax.ShapeDtypeStruct((2 * NP,), jnp.float32),
        ],
        mesh=plsc.VectorSubcoreMesh(core_axis_name="c", subcore_axis_name="s",
                                    num_cores=NC, num_subcores=NS),
        scratch_types=[
            pltpu.VMEM((NN,), jnp.float32),
            pltpu.VMEM((NN,), jnp.float32),
            pltpu.VMEM((KB, FD), jnp.float32),
            pltpu.VMEM((KB, FD), jnp.float32),
            pltpu.VMEM((CW,), jnp.int32),
            pltpu.VMEM((CW,), jnp.int32),
            pltpu.VMEM((KB,), jnp.int32),
            pltpu.VMEM((KB,), jnp.int32),
            pltpu.VMEM((KB,), jnp.float32),
            pltpu.VMEM((KB,), jnp.float32),
            pltpu.VMEM((NP // NS,), jnp.float32),
            pltpu.VMEM_SHARED((NP, FD), jnp.float32),
            pltpu.VMEM_SHARED((NP,), jnp.float32),
            pltpu.SemaphoreType.DMA,
            pltpu.SemaphoreType.DMA,
            pltpu.SemaphoreType.DMA,
            pltpu.SemaphoreType.DMA,
        ],
        compiler_params=pltpu.CompilerParams(needs_layout_passes=False),
    )


def _den_stack(denp):
    dp = jnp.stack([denp[0:NN], denp[NP:NP + NN]])
    return dp.reshape(NC, _NG, _RB).transpose(1, 0, 2)


def kernel(x, edge_index, W1, att_src1, att_dst1, b1, bn_gamma, bn_beta,
           W2, att_src2, att_dst2, b2):
    src = edge_index[0]
    dst = edge_index[1]
    b1r = b1.reshape(1, FD)
    gr = bn_gamma.reshape(1, FD)
    btr = bn_beta.reshape(1, FD)
    b2r = b2.reshape(1, FD)

    h1, s1, d1 = _tc_pre(x, W1, att_src1, att_dst1)
    outp1, denp1 = _get_sc_edge()(src, dst, s1.reshape(NN), d1.reshape(NN), h1)
    dp1 = _den_stack(denp1)
    h2, s2, d2 = _tc_mid(outp1[0:NN], outp1[NP:NP + NN], dp1, s1, d1, h1,
                         b1r, gr, btr, W2, att_src2, att_dst2)
    outp2, denp2 = _get_sc_edge()(src, dst, s2.reshape(NN), d2.reshape(NN), h2)
    dp2 = _den_stack(denp2)
    return _tc_post(outp2[0:NN], outp2[NP:NP + NN], dp2, s2, d2, h2, b2r)
